# baseline (speedup 1.0000x reference)
import jax
import jax.numpy as jnp
from jax.experimental import pallas as pl
from jax.experimental.pallas import tpu as pltpu

_N = 16
_MODES = 2
_OUT_RES = 32
_N0 = 1.0
_EY = 192
_TOTAL = 672
_CHUNKS = 4
_SPC = _N // _CHUNKS


def _make_body(p):
    def _body(u_ref, neff_ref, ey_ref, acc_in_ref, out_ref, acc_ref):
        g = pl.program_id(0)

        @pl.when(g == 0)
        def _():
            acc_ref[...] = acc_in_ref[...]

        i = p * _SPC + g
        r0 = i * _OUT_RES
        for j in range(_N):
            c = i * _N + j
            n0_ = neff_ref[c, 0]
            n1_ = neff_ref[c, 1]
            w0 = (n0_ * _N0 / (n0_ + _N0)) * u_ref[c, 0]
            w1 = (n1_ * _N0 / (n1_ + _N0)) * u_ref[c, 1]
            patch = ey_ref[0, j, 0] * w0 + ey_ref[0, j, 1] * w1
            acc_ref[pl.ds(r0, _EY), j * _OUT_RES:j * _OUT_RES + _EY] += patch

        @pl.when(g == pl.num_programs(0) - 1)
        def _():
            out_ref[...] = acc_ref[...]

    return _body


def _scatter_chunk(U, neff, chunk, acc, p):
    return pl.pallas_call(
        _make_body(p),
        grid=(_SPC,),
        in_specs=[
            pl.BlockSpec(memory_space=pltpu.SMEM),
            pl.BlockSpec(memory_space=pltpu.SMEM),
            pl.BlockSpec((1, _N, _MODES, _EY, _EY),
                         lambda g: (g, 0, 0, 0, 0)),
            pl.BlockSpec((_TOTAL, _TOTAL), lambda g: (0, 0)),
        ],
        out_specs=pl.BlockSpec((_TOTAL, _TOTAL), lambda g: (0, 0)),
        out_shape=jax.ShapeDtypeStruct((_TOTAL, _TOTAL), jnp.float32),
        scratch_shapes=[pltpu.VMEM((_TOTAL, _TOTAL), jnp.float32)],
        input_output_aliases={3: 0},
    )(U, neff, chunk, acc)


def kernel(hs, U, neff, Ey):
    del hs
    acc = jnp.zeros((_TOTAL, _TOTAL), jnp.float32)
    cells = _SPC * _N
    for p in range(_CHUNKS):
        chunk = Ey[p * cells:(p + 1) * cells].reshape(
            _SPC, _N, _MODES, _EY, _EY)
        acc = _scatter_chunk(U, neff, chunk, acc, p)
    return acc.astype(jnp.complex64)


# speedup vs baseline: 2.4391x; 2.4391x over previous
import jax
import jax.numpy as jnp
from jax.experimental import pallas as pl
from jax.experimental.pallas import tpu as pltpu

_N = 16
_MODES = 2
_OUT_RES = 32
_N0 = 1.0
_EY = 192
_TOTAL = 672


def _body(u_ref, neff_ref, ey_ref, out_ref, acc_ref):
    i = pl.program_id(0)

    @pl.when(i == 0)
    def _():
        acc_ref[...] = jnp.zeros_like(acc_ref)

    r0 = i * _OUT_RES
    for j in range(_N):
        c = i * _N + j
        n0_ = neff_ref[c, 0]
        n1_ = neff_ref[c, 1]
        w0 = (n0_ * _N0 / (n0_ + _N0)) * u_ref[c, 0]
        w1 = (n1_ * _N0 / (n1_ + _N0)) * u_ref[c, 1]
        patch = ey_ref[0, j, 0] * w0 + ey_ref[0, j, 1] * w1
        acc_ref[pl.ds(r0, _EY), j * _OUT_RES:j * _OUT_RES + _EY] += patch

    @pl.when(i == pl.num_programs(0) - 1)
    def _():
        out_ref[...] = acc_ref[...]


def kernel(hs, U, neff, Ey):
    del hs
    en = pl.pallas_call(
        _body,
        grid=(_N,),
        in_specs=[
            pl.BlockSpec(memory_space=pltpu.SMEM),
            pl.BlockSpec(memory_space=pltpu.SMEM),
            pl.BlockSpec((1, _N, _MODES, _EY, _EY),
                         lambda i: (i, 0, 0, 0, 0)),
        ],
        out_specs=pl.BlockSpec((_TOTAL, _TOTAL), lambda i: (0, 0)),
        out_shape=jax.ShapeDtypeStruct((_TOTAL, _TOTAL), jnp.float32),
        scratch_shapes=[pltpu.VMEM((_TOTAL, _TOTAL), jnp.float32)],
    )(U, neff, Ey.reshape(_N, _N, _MODES, _EY, _EY))
    return en.astype(jnp.complex64)
